# baseline (speedup 1.0000x reference)
import functools

import jax
import jax.numpy as jnp
from jax import lax
from jax.experimental import pallas as pl
from jax.experimental.pallas import tpu as pltpu
from jax.experimental.pallas import tpu_sc as plsc

V = 100000
B = 128
W = 200
PENALTY = 1.05
MAX_ID = V - 1

NC = 2
NS = 16
NW = NC * NS
ROWS_PER_W = B // NW
EPW = ROWS_PER_W * W
NIDX = (EPW + 127) // 128
EPAD = NIDX * 128

VB = 2048
GRID = (V + VB - 1) // VB


def _alpha_body(ids_ref, a_ref, ia_ref):
    ids = ids_ref[...]
    acc = jnp.zeros((B, W), jnp.float32)
    for k0 in range(0, W, 8):
        chunk = ids[:, k0:k0 + 8]
        eq = chunk[:, :, None] == ids[:, None, :]
        acc = acc + jnp.sum(eq.astype(jnp.float32), axis=1)
    cnt = jnp.where(ids >= MAX_ID, 0.0, acc)
    a = jnp.power(jnp.float32(PENALTY), cnt)
    a_ref[...] = a
    ia_ref[...] = 1.0 / a


def _transpose_body(x_ref, o_ref):
    o_ref[...] = x_ref[...].T


def _sc_penalty_body(ids_hbm, a_hbm, ia_hbm, m_ref,
                     ids_v, idx_v, a_v, ia_v, g_v, sem):
    wid = lax.axis_index("s") * NC + lax.axis_index("c")
    base = wid * EPW
    pltpu.sync_copy(ids_hbm.at[pl.ds(base, EPW)], ids_v.at[pl.ds(0, EPW)])
    pltpu.sync_copy(a_hbm.at[pl.ds(base, EPW)], a_v.at[pl.ds(0, EPW)])
    pltpu.sync_copy(ia_hbm.at[pl.ds(base, EPW)], ia_v.at[pl.ds(0, EPW)])
    lane = lax.iota(jnp.int32, 16)
    b0 = wid * ROWS_PER_W
    for j in range(EPAD // 16):
        lo = j * 16
        if lo >= EPW:
            ids_v[pl.ds(lo, 16)] = jnp.full((16,), MAX_ID, jnp.int32)
            a_v[pl.ds(lo, 16)] = jnp.full((16,), 1.0, jnp.float32)
            ia_v[pl.ds(lo, 16)] = jnp.full((16,), 1.0, jnp.float32)
            r = jnp.full((16,), 0, jnp.int32)
        else:
            r0, r1 = lo // W, (lo + 15) // W
            if lo + 15 >= EPW:
                keep = lane < EPW - lo
                ids_v[pl.ds(lo, 16)] = jnp.where(
                    keep, ids_v[pl.ds(lo, 16)], MAX_ID)
                a_v[pl.ds(lo, 16)] = jnp.where(keep, a_v[pl.ds(lo, 16)], 1.0)
                ia_v[pl.ds(lo, 16)] = jnp.where(
                    keep, ia_v[pl.ds(lo, 16)], 1.0)
                r = jnp.full((16,), r0, jnp.int32)
            elif r0 == r1:
                r = jnp.full((16,), r0, jnp.int32)
            else:
                r = jnp.where(lane < (r0 + 1) * W - lo, r0, r1)
        t = ids_v[pl.ds(lo, 16)]
        idx_v[j // 8, pl.ds((j % 8) * 16, 16)] = t * B + (b0 + r)
    gathers = [
        pltpu.async_copy(m_ref.at[idx_v.at[i]], g_v.at[i], sem)
        for i in range(NIDX)
    ]
    for cp in gathers:
        cp.wait()
    for j in range(EPAD // 16):
        g = g_v[j // 8, pl.ds((j % 8) * 16, 16)]
        a = a_v[pl.ds(j * 16, 16)]
        ia = ia_v[pl.ds(j * 16, 16)]
        g_v[j // 8, pl.ds((j % 8) * 16, 16)] = g * jnp.where(g < 0, a, ia)
    scatters = [
        pltpu.async_copy(g_v.at[i], m_ref.at[idx_v.at[i]], sem)
        for i in range(NIDX)
    ]
    for cp in scatters:
        cp.wait()


@functools.cache
def _sc_penalty():
    return functools.partial(
        pl.kernel,
        mesh=plsc.VectorSubcoreMesh(
            core_axis_name="c", subcore_axis_name="s",
            num_cores=NC, num_subcores=NS),
        scratch_types=[
            pltpu.VMEM((EPAD,), jnp.int32),
            pltpu.VMEM((NIDX, 128), jnp.int32),
            pltpu.VMEM((EPAD,), jnp.float32),
            pltpu.VMEM((EPAD,), jnp.float32),
            pltpu.VMEM((NIDX, 128), jnp.float32),
            pltpu.SemaphoreType.DMA,
        ],
    )(_sc_penalty_body)


def kernel(m_logits, input_ids, valid_len):
    start = jnp.maximum(valid_len - W, 0)
    ids = lax.dynamic_slice_in_dim(input_ids, start, W, axis=1)

    alpha, inv_alpha = pl.pallas_call(
        _alpha_body,
        in_specs=[pl.BlockSpec((B, W), lambda: (0, 0))],
        out_specs=[pl.BlockSpec((B, W), lambda: (0, 0))] * 2,
        out_shape=[jax.ShapeDtypeStruct((B, W), jnp.float32)] * 2,
    )(ids)

    m_ref = jax.new_ref(m_logits.reshape(V * B))
    _sc_penalty()(ids.reshape(B * W), alpha.reshape(B * W),
                  inv_alpha.reshape(B * W), m_ref)

    return pl.pallas_call(
        _transpose_body,
        grid=(GRID,),
        in_specs=[pl.BlockSpec((VB, B), lambda i: (i, 0))],
        out_specs=pl.BlockSpec((B, VB), lambda i: (0, i)),
        out_shape=jax.ShapeDtypeStruct((B, V), jnp.float32),
    )(m_ref[...].reshape(V, B))


# speedup vs baseline: 1.0100x; 1.0100x over previous
import functools

import jax
import jax.numpy as jnp
from jax import lax
from jax.experimental import pallas as pl
from jax.experimental.pallas import tpu as pltpu
from jax.experimental.pallas import tpu_sc as plsc

V = 100000
B = 128
W = 200
PENALTY = 1.05
MAX_ID = V - 1

NC = 2
NS = 16
NW = NC * NS
ROWS_PER_W = B // NW
EPW = ROWS_PER_W * W
NIDX = (EPW + 127) // 128
EPAD = NIDX * 128

VB = 2048
GRID = (V + VB - 1) // VB


def _alpha_body(ids_ref, a_ref, ia_ref):
    ids = ids_ref[...]
    acc = jnp.zeros((B, W), jnp.float32)
    for k0 in range(0, W, 8):
        chunk = ids[:, k0:k0 + 8]
        eq = chunk[:, :, None] == ids[:, None, :]
        acc = acc + jnp.sum(eq.astype(jnp.float32), axis=1)
    cnt = jnp.where(ids >= MAX_ID, 0.0, acc)
    a = jnp.power(jnp.float32(PENALTY), cnt)
    a_ref[...] = a
    ia_ref[...] = 1.0 / a


def _transpose_body(x_ref, o_ref):
    o_ref[...] = x_ref[...].T


def _sc_penalty_body(morig_hbm, ids_hbm, a_hbm, ia_hbm, m_ref,
                     ids_v, idx_v, a_v, ia_v, g_v, sem):
    wid = lax.axis_index("s") * NC + lax.axis_index("c")
    base = wid * EPW
    pltpu.sync_copy(ids_hbm.at[pl.ds(base, EPW)], ids_v.at[pl.ds(0, EPW)])
    pltpu.sync_copy(a_hbm.at[pl.ds(base, EPW)], a_v.at[pl.ds(0, EPW)])
    pltpu.sync_copy(ia_hbm.at[pl.ds(base, EPW)], ia_v.at[pl.ds(0, EPW)])
    lane = lax.iota(jnp.int32, 16)
    b0 = wid * ROWS_PER_W
    for j in range(EPAD // 16):
        lo = j * 16
        if lo >= EPW:
            ids_v[pl.ds(lo, 16)] = jnp.full((16,), MAX_ID, jnp.int32)
            a_v[pl.ds(lo, 16)] = jnp.full((16,), 1.0, jnp.float32)
            ia_v[pl.ds(lo, 16)] = jnp.full((16,), 1.0, jnp.float32)
            r = jnp.full((16,), 0, jnp.int32)
        else:
            r0, r1 = lo // W, (lo + 15) // W
            if lo + 15 >= EPW:
                keep = lane < EPW - lo
                ids_v[pl.ds(lo, 16)] = jnp.where(
                    keep, ids_v[pl.ds(lo, 16)], MAX_ID)
                a_v[pl.ds(lo, 16)] = jnp.where(keep, a_v[pl.ds(lo, 16)], 1.0)
                ia_v[pl.ds(lo, 16)] = jnp.where(
                    keep, ia_v[pl.ds(lo, 16)], 1.0)
                r = jnp.full((16,), r0, jnp.int32)
            elif r0 == r1:
                r = jnp.full((16,), r0, jnp.int32)
            else:
                r = jnp.where(lane < (r0 + 1) * W - lo, r0, r1)
        t = ids_v[pl.ds(lo, 16)]
        idx_v[j // 8, pl.ds((j % 8) * 16, 16)] = t * B + (b0 + r)
    gathers = [
        pltpu.async_copy(morig_hbm.at[idx_v.at[i]], g_v.at[i], sem)
        for i in range(NIDX)
    ]
    for cp in gathers:
        cp.wait()
    for j in range(EPAD // 16):
        g = g_v[j // 8, pl.ds((j % 8) * 16, 16)]
        a = a_v[pl.ds(j * 16, 16)]
        ia = ia_v[pl.ds(j * 16, 16)]
        g_v[j // 8, pl.ds((j % 8) * 16, 16)] = g * jnp.where(g < 0, a, ia)
    scatters = [
        pltpu.async_copy(g_v.at[i], m_ref.at[idx_v.at[i]], sem)
        for i in range(NIDX)
    ]
    for cp in scatters:
        cp.wait()


@functools.cache
def _sc_penalty():
    return functools.partial(
        pl.kernel,
        mesh=plsc.VectorSubcoreMesh(
            core_axis_name="c", subcore_axis_name="s",
            num_cores=NC, num_subcores=NS),
        scratch_types=[
            pltpu.VMEM((EPAD,), jnp.int32),
            pltpu.VMEM((NIDX, 128), jnp.int32),
            pltpu.VMEM((EPAD,), jnp.float32),
            pltpu.VMEM((EPAD,), jnp.float32),
            pltpu.VMEM((NIDX, 128), jnp.float32),
            pltpu.SemaphoreType.DMA,
        ],
    )(_sc_penalty_body)


def kernel(m_logits, input_ids, valid_len):
    start = jnp.maximum(valid_len - W, 0)
    ids = lax.dynamic_slice_in_dim(input_ids, start, W, axis=1)

    alpha, inv_alpha = pl.pallas_call(
        _alpha_body,
        in_specs=[pl.BlockSpec((B, W), lambda: (0, 0))],
        out_specs=[pl.BlockSpec((B, W), lambda: (0, 0))] * 2,
        out_shape=[jax.ShapeDtypeStruct((B, W), jnp.float32)] * 2,
    )(ids)

    m_ref = jax.new_ref(m_logits.reshape(V * B))
    _sc_penalty()(m_logits.reshape(V * B), ids.reshape(B * W),
                  alpha.reshape(B * W), inv_alpha.reshape(B * W), m_ref)

    return pl.pallas_call(
        _transpose_body,
        grid=(GRID,),
        in_specs=[pl.BlockSpec((VB, B), lambda i: (i, 0))],
        out_specs=pl.BlockSpec((B, VB), lambda i: (0, i)),
        out_shape=jax.ShapeDtypeStruct((B, V), jnp.float32),
    )(m_ref[...].reshape(V, B))


# speedup vs baseline: 1.9083x; 1.8893x over previous
import functools
import math

import jax
import jax.numpy as jnp
from jax import lax
from jax.experimental import pallas as pl
from jax.experimental.pallas import tpu as pltpu
from jax.experimental.pallas import tpu_sc as plsc

V = 100000
B = 128
W = 200
PW = 256
PENALTY = 1.05
MAX_ID = V - 1
LN_P = math.log(PENALTY)

NC = 2
NS = 16
NW = NC * NS
ROWS_PER_W = B // NW

VB = 2048
GRID = (V + VB - 1) // VB


def _transpose_body(x_ref, o_ref):
    o_ref[...] = x_ref[...].T


def _counts_body(ids_ref, c_ref):
    ids = ids_ref[...]
    acc = jnp.zeros((B, W), jnp.float32)
    for k0 in range(0, W, 8):
        chunk = ids[:, k0:k0 + 8]
        eq = chunk[:, :, None] == ids[:, None, :]
        acc = acc + jnp.sum(eq.astype(jnp.float32), axis=1)
    cnt = jnp.where(ids >= MAX_ID, 0.0, acc)
    c_ref[:, :W] = cnt
    c_ref[:, W:] = jnp.zeros((B, PW - W), jnp.float32)


def _sc_penalty_body(mflat, ids_hbm, counts_hbm, out_ref,
                     ids_v, gidx_v, sidx_v, g_v, val_v, c_v, sem):
    wid = lax.axis_index("s") * NC + lax.axis_index("c")
    lane = lax.iota(jnp.int32, 16)
    for r in range(ROWS_PER_W):
        b = wid * ROWS_PER_W + r
        pltpu.sync_copy(ids_hbm.at[pl.ds(b * W, W)], ids_v.at[pl.ds(0, W)])
        pltpu.sync_copy(counts_hbm.at[pl.ds(b * PW, PW)], c_v)
        tail = ids_v[pl.ds(192, 16)]
        ids_v[pl.ds(192, 16)] = jnp.where(lane < W - 192, tail, MAX_ID)
        for q in range(13, PW // 16):
            ids_v[pl.ds(q * 16, 16)] = jnp.full((16,), MAX_ID, jnp.int32)
        for j in range(PW // 16):
            t = ids_v[pl.ds(j * 16, 16)]
            gidx_v[j // 8, pl.ds((j % 8) * 16, 16)] = t * B + b
            sidx_v[j // 8, pl.ds((j % 8) * 16, 16)] = b * V + t
        cp0 = pltpu.async_copy(mflat.at[gidx_v.at[0]], g_v.at[0], sem)
        cp1 = pltpu.async_copy(mflat.at[gidx_v.at[1]], g_v.at[1], sem)
        cp0.wait()
        cp1.wait()
        for j in range(PW // 16):
            g = g_v[j // 8, pl.ds((j % 8) * 16, 16)]
            c = c_v[pl.ds(j * 16, 16)]
            a = jnp.exp(c * LN_P)
            val_v[j // 8, pl.ds((j % 8) * 16, 16)] = jnp.where(
                g < 0, g * a, g / a)
        sc0 = pltpu.async_copy(val_v.at[0], out_ref.at[sidx_v.at[0]], sem)
        sc1 = pltpu.async_copy(val_v.at[1], out_ref.at[sidx_v.at[1]], sem)
        sc0.wait()
        sc1.wait()


@functools.cache
def _sc_penalty():
    return functools.partial(
        pl.kernel,
        mesh=plsc.VectorSubcoreMesh(
            core_axis_name="c", subcore_axis_name="s",
            num_cores=NC, num_subcores=NS),
        scratch_types=[
            pltpu.VMEM((PW,), jnp.int32),
            pltpu.VMEM((2, 128), jnp.int32),
            pltpu.VMEM((2, 128), jnp.int32),
            pltpu.VMEM((2, 128), jnp.float32),
            pltpu.VMEM((2, 128), jnp.float32),
            pltpu.VMEM((PW,), jnp.float32),
            pltpu.SemaphoreType.DMA,
        ],
    )(_sc_penalty_body)


def kernel(m_logits, input_ids, valid_len):
    start = jnp.maximum(valid_len - W, 0)
    ids = lax.dynamic_slice_in_dim(input_ids, start, W, axis=1)

    out_t = pl.pallas_call(
        _transpose_body,
        grid=(GRID,),
        in_specs=[pl.BlockSpec((VB, B), lambda i: (i, 0))],
        out_specs=pl.BlockSpec((B, VB), lambda i: (0, i)),
        out_shape=jax.ShapeDtypeStruct((B, V), jnp.float32),
    )(m_logits)

    counts = pl.pallas_call(
        _counts_body,
        in_specs=[pl.BlockSpec((B, W), lambda: (0, 0))],
        out_specs=pl.BlockSpec((B, PW), lambda: (0, 0)),
        out_shape=jax.ShapeDtypeStruct((B, PW), jnp.float32),
    )(ids)

    out_ref = jax.new_ref(out_t.reshape(B * V))
    _sc_penalty()(m_logits.reshape(V * B), ids.reshape(B * W),
                  counts.reshape(B * PW), out_ref)
    return out_ref[...].reshape(B, V)
